# baseline (speedup 1.0000x reference)
import jax
import jax.numpy as jnp
from jax import lax
from jax.experimental import pallas as pl
from jax.experimental.pallas import tpu as pltpu
from jax.experimental.pallas import tpu_sc as plsc

N = 10000
E = 320000
F = 128
H = 256
S = 32
NG = 8
L = 128

NC = 2
NS = 16
NW = NC * NS
HH = H // 2
EPW = E // NW
EPT = E // NS
C = 80
GCH = EPW // C
SCH = EPT // C
RPT = 624
ZR = 78

_SC_PARAMS = pltpu.CompilerParams(use_tc_tiling_on_sc=False,
                                  needs_layout_passes=False)


def _dg(x, w):
    return lax.dot_general(x, w, (((1,), (1,)), ((), ())),
                           preferred_element_type=jnp.float32)


def _make_gather(tn):
    mesh = plsc.VectorSubcoreMesh(core_axis_name="c", subcore_axis_name="s")
    out_type = (jax.ShapeDtypeStruct((E, HH), jnp.float32),
                jax.ShapeDtypeStruct((E, HH), jnp.float32))
    scratch = [
        pltpu.VMEM((EPW,), jnp.int32),
        pltpu.VMEM((EPW,), jnp.int32),
        pltpu.VMEM((C,), jnp.int32),
        pltpu.VMEM((C,), jnp.int32),
        pltpu.VMEM((C, HH), jnp.float32),
        pltpu.VMEM((C, HH), jnp.float32),
        pltpu.SemaphoreType.DMA,
    ]

    def body(table, idx1, idx2, o1, o2, i1_v, i2_v, c1_v, c2_v, b1, b2, sem):
        cid = lax.axis_index("c")
        sid = lax.axis_index("s")
        base = (cid * NS + sid) * EPW
        pltpu.sync_copy(idx1.at[pl.ds(base, EPW)], i1_v)
        pltpu.sync_copy(idx2.at[pl.ds(base, EPW)], i2_v)

        def chunk(j, _):
            cb = j * C
            for k in range(C // 16):
                o = pl.ds(k * 16, 16)
                c1_v[o] = i1_v[pl.ds(cb + k * 16, 16)]
                c2_v[o] = i2_v[pl.ds(cb + k * 16, 16)]
            pltpu.async_copy(table.at[c1_v], b1, sem).wait()
            pltpu.sync_copy(b1, o1.at[pl.ds(base + cb, C)])
            pltpu.async_copy(table.at[c2_v], b2, sem).wait()
            pltpu.sync_copy(b2, o2.at[pl.ds(base + cb, C)])
            return 0

        lax.fori_loop(0, GCH, chunk, 0)

    return pl.kernel(body, out_type=out_type, mesh=mesh,
                     scratch_types=scratch, compiler_params=_SC_PARAMS)


_gather_x = _make_gather(N)
_gather_h = _make_gather(2 * N)


def _make_scatter():
    mesh = plsc.VectorSubcoreMesh(core_axis_name="c", subcore_axis_name="s")
    out_type = (jax.ShapeDtypeStruct((NC * N, HH), jnp.float32),)
    scratch = [
        pltpu.VMEM((EPT,), jnp.int32),
        pltpu.VMEM((C,), jnp.int32),
        pltpu.VMEM((C, HH), jnp.float32),
        pltpu.VMEM((ZR, HH), jnp.float32),
        pltpu.VMEM_SHARED((N, HH), jnp.float32),
        pltpu.SemaphoreType.DMA,
    ]

    def body(mlo, mhi, dst, acc_out, dst_v, dc_v, buf, zbuf, acc_sh, sem):
        cid = lax.axis_index("c")
        sid = lax.axis_index("s")
        zvec = jnp.zeros((16,), jnp.float32)

        def zloop(i, _):
            for r in range(HH // 16):
                zbuf[i, pl.ds(r * 16, 16)] = zvec
            return 0

        lax.fori_loop(0, ZR, zloop, 0)
        row0 = sid * RPT
        for k in range(RPT // ZR):
            pltpu.sync_copy(zbuf, acc_sh.at[pl.ds(row0 + k * ZR, ZR)])

        tail0 = NS * RPT
        ntail = N - tail0

        @pl.when(sid == NS - 1)
        def _():
            pltpu.sync_copy(zbuf.at[pl.ds(0, ntail)],
                            acc_sh.at[pl.ds(tail0, ntail)])

        base = sid * EPT
        pltpu.sync_copy(dst.at[pl.ds(base, EPT)], dst_v)

        plsc.subcore_barrier()

        def chunk(j, _):
            cb = j * C
            for k in range(C // 16):
                dc_v[pl.ds(k * 16, 16)] = dst_v[pl.ds(cb + k * 16, 16)]

            @pl.when(cid == 0)
            def _():
                pltpu.sync_copy(mlo.at[pl.ds(base + cb, C)], buf)

            @pl.when(cid == 1)
            def _():
                pltpu.sync_copy(mhi.at[pl.ds(base + cb, C)], buf)

            pltpu.sync_copy(buf, acc_sh.at[dc_v], add=True)
            return 0

        lax.fori_loop(0, SCH, chunk, 0)

        plsc.subcore_barrier()

        pltpu.sync_copy(acc_sh.at[pl.ds(row0, RPT)],
                        acc_out.at[pl.ds(cid * N + row0, RPT)])

        @pl.when(sid == NS - 1)
        def _():
            pltpu.sync_copy(acc_sh.at[pl.ds(tail0, ntail)],
                            acc_out.at[pl.ds(cid * N + tail0, ntail)])

    return pl.kernel(body, out_type=out_type, mesh=mesh,
                     scratch_types=scratch, compiler_params=_SC_PARAMS)


_scatter = _make_scatter()


BE = 2000
GE = E // BE
BN = 1000
G = N // BN


def _m1_body(xi_ref, xj_ref, w1_ref, b1_ref, w2_ref, b2_ref, lo_ref, hi_ref):
    tmp = jnp.concatenate([xi_ref[...], xj_ref[...]], axis=1)
    h = jnp.maximum(_dg(tmp, w1_ref[...]) + b1_ref[...], 0.0)
    m = _dg(h, w2_ref[...]) + b2_ref[...]
    lo_ref[...] = m[:, :HH]
    hi_ref[...] = m[:, HH:]


def _m1(xi, xj, w1, b1, w2, b2):
    return pl.pallas_call(
        _m1_body,
        grid=(GE,),
        in_specs=[
            pl.BlockSpec((BE, F), lambda i: (i, 0)),
            pl.BlockSpec((BE, F), lambda i: (i, 0)),
            pl.BlockSpec((H, 2 * F), lambda i: (0, 0)),
            pl.BlockSpec((1, H), lambda i: (0, 0)),
            pl.BlockSpec((H, H), lambda i: (0, 0)),
            pl.BlockSpec((1, H), lambda i: (0, 0)),
        ],
        out_specs=[
            pl.BlockSpec((BE, HH), lambda i: (i, 0)),
            pl.BlockSpec((BE, HH), lambda i: (i, 0)),
        ],
        out_shape=[
            jax.ShapeDtypeStruct((E, HH), jnp.float32),
            jax.ShapeDtypeStruct((E, HH), jnp.float32),
        ],
    )(xi, xj, w1, b1, w2, b2)


def _m2_body(il_ref, ih_ref, jl_ref, jh_ref, w1_ref, b1_ref, w2_ref, b2_ref,
             lo_ref, hi_ref):
    tmp = jnp.maximum(jnp.concatenate(
        [il_ref[...], ih_ref[...], jl_ref[...], jh_ref[...]], axis=1), 0.0)
    h = jnp.maximum(_dg(tmp, w1_ref[...]) + b1_ref[...], 0.0)
    m = _dg(h, w2_ref[...]) + b2_ref[...]
    lo_ref[...] = m[:, :HH]
    hi_ref[...] = m[:, HH:]


def _m2(il, ih, jl, jh, w1, b1, w2, b2):
    return pl.pallas_call(
        _m2_body,
        grid=(GE,),
        in_specs=[
            pl.BlockSpec((BE, HH), lambda i: (i, 0)),
            pl.BlockSpec((BE, HH), lambda i: (i, 0)),
            pl.BlockSpec((BE, HH), lambda i: (i, 0)),
            pl.BlockSpec((BE, HH), lambda i: (i, 0)),
            pl.BlockSpec((H, 2 * H), lambda i: (0, 0)),
            pl.BlockSpec((1, H), lambda i: (0, 0)),
            pl.BlockSpec((H, H), lambda i: (0, 0)),
            pl.BlockSpec((1, H), lambda i: (0, 0)),
        ],
        out_specs=[
            pl.BlockSpec((BE, HH), lambda i: (i, 0)),
            pl.BlockSpec((BE, HH), lambda i: (i, 0)),
        ],
        out_shape=[
            jax.ShapeDtypeStruct((E, HH), jnp.float32),
            jax.ShapeDtypeStruct((E, HH), jnp.float32),
        ],
    )(il, ih, jl, jh, w1, b1, w2, b2)


def _tc3_body(q0_ref, q1_ref, batch_ref,
              pw1_ref, pb1_ref, pw2_ref, pb2_ref,
              ow1_ref, ob1_ref, ow2_ref, ob2_ref,
              s_ref, lat_ref, pacc):
    i = pl.program_id(0)
    h = jnp.maximum(jnp.concatenate([q0_ref[...], q1_ref[...]], axis=1), 0.0)
    t = jnp.maximum(_dg(h, pw1_ref[...]) + pb1_ref[...], 0.0)
    a = _dg(t, pw2_ref[...]) + pb2_ref[...]
    m = jnp.max(a, axis=-1, keepdims=True)
    ex = jnp.exp(a - m)
    sm = ex / jnp.sum(ex, axis=-1, keepdims=True)
    s_ref[...] = sm
    gi = lax.broadcasted_iota(jnp.int32, (BN, NG), 1)
    gm = (batch_ref[...] == gi).astype(jnp.float32)
    srep = jnp.concatenate([sm] * NG, axis=1)
    grep = jnp.concatenate(
        [jnp.broadcast_to(gm[:, g:g + 1], (BN, S)) for g in range(NG)],
        axis=1)
    w_assign = srep * grep

    @pl.when(i == 0)
    def _():
        pacc[...] = jnp.zeros((NG * S, H), jnp.float32)

    pacc[...] += lax.dot_general(w_assign, h, (((0,), (0,)), ((), ())),
                                 precision=lax.Precision.HIGHEST,
                                 preferred_element_type=jnp.float32)

    @pl.when(i == G - 1)
    def _():
        p = pacc[...]
        t2 = jnp.maximum(_dg(p, ow1_ref[...]) + ob1_ref[...], 0.0)
        lat_ref[...] = _dg(t2, ow2_ref[...]) + ob2_ref[...]


def _tc3(accf, batch2, pw1, pb1, pw2, pb2, ow1, ob1, ow2, ob2):
    return pl.pallas_call(
        _tc3_body,
        grid=(G,),
        in_specs=[
            pl.BlockSpec((BN, HH), lambda i: (i, 0)),
            pl.BlockSpec((BN, HH), lambda i: (G + i, 0)),
            pl.BlockSpec((BN, 1), lambda i: (i, 0)),
            pl.BlockSpec((H, H), lambda i: (0, 0)),
            pl.BlockSpec((1, H), lambda i: (0, 0)),
            pl.BlockSpec((S, H), lambda i: (0, 0)),
            pl.BlockSpec((1, S), lambda i: (0, 0)),
            pl.BlockSpec((H, H), lambda i: (0, 0)),
            pl.BlockSpec((1, H), lambda i: (0, 0)),
            pl.BlockSpec((L, H), lambda i: (0, 0)),
            pl.BlockSpec((1, L), lambda i: (0, 0)),
        ],
        out_specs=[
            pl.BlockSpec((BN, S), lambda i: (i, 0)),
            pl.BlockSpec((NG * S, L), lambda i: (0, 0)),
        ],
        out_shape=[
            jax.ShapeDtypeStruct((N, S), jnp.float32),
            jax.ShapeDtypeStruct((NG * S, L), jnp.float32),
        ],
        scratch_shapes=[pltpu.VMEM((NG * S, H), jnp.float32)],
    )(accf, accf, batch2, pw1, pb1, pw2, pb2, ow1, ob1, ow2, ob2)


def kernel(x, edge_index, batch,
           g1w1, g1b1, g1w2, g1b2,
           g2w1, g2b1, g2w2, g2b2,
           pw1, pb1, pw2, pb2,
           ow1, ob1, ow2, ob2):
    ei = edge_index.astype(jnp.int32)
    src = ei[0]
    dst = ei[1]
    srcN = src + N
    dstN = dst + N
    batch2 = batch.astype(jnp.int32).reshape(N, 1)

    xi, xj = _gather_x(x, dst, src)
    mlo1, mhi1 = _m1(xi, xj, g1w1, g1b1.reshape(1, H), g1w2, g1b2.reshape(1, H))
    (acc1,) = _scatter(mlo1, mhi1, dst)

    il, jl = _gather_h(acc1, dst, src)
    ih, jh = _gather_h(acc1, dstN, srcN)
    mlo2, mhi2 = _m2(il, ih, jl, jh, g2w1, g2b1.reshape(1, H),
                     g2w2, g2b2.reshape(1, H))
    (acc2,) = _scatter(mlo2, mhi2, dst)

    s, lat = _tc3(acc2, batch2,
                  pw1, pb1.reshape(1, H), pw2, pb2.reshape(1, S),
                  ow1, ob1.reshape(1, H), ow2, ob2.reshape(1, L))
    return lat.reshape(NG, S, L), s


# speedup vs baseline: 1.3869x; 1.3869x over previous
import jax
import jax.numpy as jnp
from jax import lax
from jax.experimental import pallas as pl
from jax.experimental.pallas import tpu as pltpu
from jax.experimental.pallas import tpu_sc as plsc

N = 10000
E = 320000
F = 128
H = 256
S = 32
NG = 8
L = 128

NC = 2
NS = 16
NW = NC * NS
HH = H // 2
EPW = E // NW
EPT = E // NS
C = 80
GCH = EPW // C
SCH = EPT // C
RPT = 624
ZR = 78

_SC_PARAMS = pltpu.CompilerParams(use_tc_tiling_on_sc=False,
                                  needs_layout_passes=False)


def _dg(x, w):
    return lax.dot_general(x, w, (((1,), (1,)), ((), ())),
                           preferred_element_type=jnp.float32)


def _make_gather(p):
    mesh = plsc.VectorSubcoreMesh(core_axis_name="c", subcore_axis_name="s")
    out_type = tuple(jax.ShapeDtypeStruct((E, HH), jnp.float32)
                     for _ in range(p))
    scratch = ([pltpu.VMEM((EPW,), jnp.int32) for _ in range(p)]
               + [pltpu.VMEM((C,), jnp.int32) for _ in range(p)]
               + [pltpu.VMEM((C, HH), jnp.float32) for _ in range(p)]
               + [pltpu.SemaphoreType.DMA, pltpu.SemaphoreType.DMA])

    def body(table, *rest):
        idxs = rest[:p]
        outs = rest[p:2 * p]
        ivs = rest[2 * p:3 * p]
        cvs = rest[3 * p:4 * p]
        bufs = rest[4 * p:5 * p]
        semg, semw = rest[5 * p], rest[5 * p + 1]
        cid = lax.axis_index("c")
        sid = lax.axis_index("s")
        base = (cid * NS + sid) * EPW
        for q in range(p):
            pltpu.sync_copy(idxs[q].at[pl.ds(base, EPW)], ivs[q])

        def chunk(j, _):
            cb = j * C
            for q in range(p):
                for k in range(C // 16):
                    cvs[q][pl.ds(k * 16, 16)] = ivs[q][pl.ds(cb + k * 16, 16)]
            gps = [pltpu.async_copy(table.at[cvs[q]], bufs[q], semg)
                   for q in range(p)]
            for gp in gps:
                gp.wait()
            wps = [pltpu.async_copy(bufs[q], outs[q].at[pl.ds(base + cb, C)],
                                    semw)
                   for q in range(p)]
            for wp in wps:
                wp.wait()
            return 0

        lax.fori_loop(0, GCH, chunk, 0)

    return pl.kernel(body, out_type=out_type, mesh=mesh,
                     scratch_types=scratch, compiler_params=_SC_PARAMS)


_gather_x = _make_gather(2)
_gather_h = _make_gather(4)


def _make_scatter():
    mesh = plsc.VectorSubcoreMesh(core_axis_name="c", subcore_axis_name="s")
    out_type = (jax.ShapeDtypeStruct((NC * N, HH), jnp.float32),)
    scratch = [
        pltpu.VMEM((EPT,), jnp.int32),
        pltpu.VMEM((C,), jnp.int32),
        pltpu.VMEM((C,), jnp.int32),
        pltpu.VMEM((C, HH), jnp.float32),
        pltpu.VMEM((C, HH), jnp.float32),
        pltpu.VMEM((ZR, HH), jnp.float32),
        pltpu.VMEM_SHARED((N, HH), jnp.float32),
        pltpu.SemaphoreType.DMA,
        pltpu.SemaphoreType.DMA,
    ]

    def body(mlo, mhi, dst, acc_out, dst_v, dc0, dc1, buf0, buf1, zbuf,
             acc_sh, sem0, sem1):
        cid = lax.axis_index("c")
        sid = lax.axis_index("s")
        zvec = jnp.zeros((16,), jnp.float32)

        def zloop(i, _):
            for r in range(HH // 16):
                zbuf[i, pl.ds(r * 16, 16)] = zvec
            return 0

        lax.fori_loop(0, ZR, zloop, 0)
        row0 = sid * RPT
        for k in range(RPT // ZR):
            pltpu.sync_copy(zbuf, acc_sh.at[pl.ds(row0 + k * ZR, ZR)])

        tail0 = NS * RPT
        ntail = N - tail0

        @pl.when(sid == NS - 1)
        def _():
            pltpu.sync_copy(zbuf.at[pl.ds(0, ntail)],
                            acc_sh.at[pl.ds(tail0, ntail)])

        base = sid * EPT
        pltpu.sync_copy(dst.at[pl.ds(base, EPT)], dst_v)

        plsc.subcore_barrier()

        def fire_load(cb, buf, sem):
            @pl.when(cid == 0)
            def _():
                pltpu.async_copy(mlo.at[pl.ds(base + cb, C)], buf, sem)

            @pl.when(cid == 1)
            def _():
                pltpu.async_copy(mhi.at[pl.ds(base + cb, C)], buf, sem)

        def drain(buf, sem):
            pltpu.make_async_copy(mlo.at[pl.ds(0, C)], buf, sem).wait()

        def refresh(dc, cb):
            for k in range(C // 16):
                dc[pl.ds(k * 16, 16)] = dst_v[pl.ds(cb + k * 16, 16)]

        fire_load(0, buf0, sem0)

        def chunk2(j2, _):
            ca = 2 * j2 * C
            cb2 = (2 * j2 + 1) * C
            fire_load(cb2, buf1, sem1)
            drain(buf0, sem0)
            refresh(dc0, ca)
            pltpu.sync_copy(buf0, acc_sh.at[dc0], add=True)

            @pl.when(j2 < SCH // 2 - 1)
            def _():
                fire_load((2 * j2 + 2) * C, buf0, sem0)

            drain(buf1, sem1)
            refresh(dc1, cb2)
            pltpu.sync_copy(buf1, acc_sh.at[dc1], add=True)
            return 0

        lax.fori_loop(0, SCH // 2, chunk2, 0)

        plsc.subcore_barrier()

        pltpu.sync_copy(acc_sh.at[pl.ds(row0, RPT)],
                        acc_out.at[pl.ds(cid * N + row0, RPT)])

        @pl.when(sid == NS - 1)
        def _():
            pltpu.sync_copy(acc_sh.at[pl.ds(tail0, ntail)],
                            acc_out.at[pl.ds(cid * N + tail0, ntail)])

    return pl.kernel(body, out_type=out_type, mesh=mesh,
                     scratch_types=scratch, compiler_params=_SC_PARAMS)


_scatter = _make_scatter()


BE = 2000
GE = E // BE
BN = 1000
G = N // BN


def _m1_body(xi_ref, xj_ref, w1_ref, b1_ref, w2_ref, b2_ref, lo_ref, hi_ref):
    tmp = jnp.concatenate([xi_ref[...], xj_ref[...]], axis=1)
    h = jnp.maximum(_dg(tmp, w1_ref[...]) + b1_ref[...], 0.0)
    m = _dg(h, w2_ref[...]) + b2_ref[...]
    lo_ref[...] = m[:, :HH]
    hi_ref[...] = m[:, HH:]


def _m1(xi, xj, w1, b1, w2, b2):
    return pl.pallas_call(
        _m1_body,
        grid=(GE,),
        in_specs=[
            pl.BlockSpec((BE, F), lambda i: (i, 0)),
            pl.BlockSpec((BE, F), lambda i: (i, 0)),
            pl.BlockSpec((H, 2 * F), lambda i: (0, 0)),
            pl.BlockSpec((1, H), lambda i: (0, 0)),
            pl.BlockSpec((H, H), lambda i: (0, 0)),
            pl.BlockSpec((1, H), lambda i: (0, 0)),
        ],
        out_specs=[
            pl.BlockSpec((BE, HH), lambda i: (i, 0)),
            pl.BlockSpec((BE, HH), lambda i: (i, 0)),
        ],
        out_shape=[
            jax.ShapeDtypeStruct((E, HH), jnp.float32),
            jax.ShapeDtypeStruct((E, HH), jnp.float32),
        ],
    )(xi, xj, w1, b1, w2, b2)


def _m2_body(il_ref, ih_ref, jl_ref, jh_ref, w1_ref, b1_ref, w2_ref, b2_ref,
             lo_ref, hi_ref):
    tmp = jnp.maximum(jnp.concatenate(
        [il_ref[...], ih_ref[...], jl_ref[...], jh_ref[...]], axis=1), 0.0)
    h = jnp.maximum(_dg(tmp, w1_ref[...]) + b1_ref[...], 0.0)
    m = _dg(h, w2_ref[...]) + b2_ref[...]
    lo_ref[...] = m[:, :HH]
    hi_ref[...] = m[:, HH:]


def _m2(il, ih, jl, jh, w1, b1, w2, b2):
    return pl.pallas_call(
        _m2_body,
        grid=(GE,),
        in_specs=[
            pl.BlockSpec((BE, HH), lambda i: (i, 0)),
            pl.BlockSpec((BE, HH), lambda i: (i, 0)),
            pl.BlockSpec((BE, HH), lambda i: (i, 0)),
            pl.BlockSpec((BE, HH), lambda i: (i, 0)),
            pl.BlockSpec((H, 2 * H), lambda i: (0, 0)),
            pl.BlockSpec((1, H), lambda i: (0, 0)),
            pl.BlockSpec((H, H), lambda i: (0, 0)),
            pl.BlockSpec((1, H), lambda i: (0, 0)),
        ],
        out_specs=[
            pl.BlockSpec((BE, HH), lambda i: (i, 0)),
            pl.BlockSpec((BE, HH), lambda i: (i, 0)),
        ],
        out_shape=[
            jax.ShapeDtypeStruct((E, HH), jnp.float32),
            jax.ShapeDtypeStruct((E, HH), jnp.float32),
        ],
    )(il, ih, jl, jh, w1, b1, w2, b2)


def _tc3_body(q0_ref, q1_ref, batch_ref,
              pw1_ref, pb1_ref, pw2_ref, pb2_ref,
              ow1_ref, ob1_ref, ow2_ref, ob2_ref,
              s_ref, lat_ref, pacc):
    i = pl.program_id(0)
    h = jnp.maximum(jnp.concatenate([q0_ref[...], q1_ref[...]], axis=1), 0.0)
    t = jnp.maximum(_dg(h, pw1_ref[...]) + pb1_ref[...], 0.0)
    a = _dg(t, pw2_ref[...]) + pb2_ref[...]
    m = jnp.max(a, axis=-1, keepdims=True)
    ex = jnp.exp(a - m)
    sm = ex / jnp.sum(ex, axis=-1, keepdims=True)
    s_ref[...] = sm
    gi = lax.broadcasted_iota(jnp.int32, (BN, NG), 1)
    gm = (batch_ref[...] == gi).astype(jnp.float32)
    srep = jnp.concatenate([sm] * NG, axis=1)
    grep = jnp.concatenate(
        [jnp.broadcast_to(gm[:, g:g + 1], (BN, S)) for g in range(NG)],
        axis=1)
    w_assign = srep * grep

    @pl.when(i == 0)
    def _():
        pacc[...] = jnp.zeros((NG * S, H), jnp.float32)

    pacc[...] += lax.dot_general(w_assign, h, (((0,), (0,)), ((), ())),
                                 precision=lax.Precision.HIGHEST,
                                 preferred_element_type=jnp.float32)

    @pl.when(i == G - 1)
    def _():
        p = pacc[...]
        t2 = jnp.maximum(_dg(p, ow1_ref[...]) + ob1_ref[...], 0.0)
        lat_ref[...] = _dg(t2, ow2_ref[...]) + ob2_ref[...]


def _tc3(accf, batch2, pw1, pb1, pw2, pb2, ow1, ob1, ow2, ob2):
    return pl.pallas_call(
        _tc3_body,
        grid=(G,),
        in_specs=[
            pl.BlockSpec((BN, HH), lambda i: (i, 0)),
            pl.BlockSpec((BN, HH), lambda i: (G + i, 0)),
            pl.BlockSpec((BN, 1), lambda i: (i, 0)),
            pl.BlockSpec((H, H), lambda i: (0, 0)),
            pl.BlockSpec((1, H), lambda i: (0, 0)),
            pl.BlockSpec((S, H), lambda i: (0, 0)),
            pl.BlockSpec((1, S), lambda i: (0, 0)),
            pl.BlockSpec((H, H), lambda i: (0, 0)),
            pl.BlockSpec((1, H), lambda i: (0, 0)),
            pl.BlockSpec((L, H), lambda i: (0, 0)),
            pl.BlockSpec((1, L), lambda i: (0, 0)),
        ],
        out_specs=[
            pl.BlockSpec((BN, S), lambda i: (i, 0)),
            pl.BlockSpec((NG * S, L), lambda i: (0, 0)),
        ],
        out_shape=[
            jax.ShapeDtypeStruct((N, S), jnp.float32),
            jax.ShapeDtypeStruct((NG * S, L), jnp.float32),
        ],
        scratch_shapes=[pltpu.VMEM((NG * S, H), jnp.float32)],
    )(accf, accf, batch2, pw1, pb1, pw2, pb2, ow1, ob1, ow2, ob2)


def kernel(x, edge_index, batch,
           g1w1, g1b1, g1w2, g1b2,
           g2w1, g2b1, g2w2, g2b2,
           pw1, pb1, pw2, pb2,
           ow1, ob1, ow2, ob2):
    ei = edge_index.astype(jnp.int32)
    src = ei[0]
    dst = ei[1]
    srcN = src + N
    dstN = dst + N
    batch2 = batch.astype(jnp.int32).reshape(N, 1)

    xi, xj = _gather_x(x, dst, src)
    mlo1, mhi1 = _m1(xi, xj, g1w1, g1b1.reshape(1, H), g1w2, g1b2.reshape(1, H))
    (acc1,) = _scatter(mlo1, mhi1, dst)

    il, ih, jl, jh = _gather_h(acc1, dst, dstN, src, srcN)
    mlo2, mhi2 = _m2(il, ih, jl, jh, g2w1, g2b1.reshape(1, H),
                     g2w2, g2b2.reshape(1, H))
    (acc2,) = _scatter(mlo2, mhi2, dst)

    s, lat = _tc3(acc2, batch2,
                  pw1, pb1.reshape(1, H), pw2, pb2.reshape(1, S),
                  ow1, ob1.reshape(1, H), ow2, ob2.reshape(1, L))
    return lat.reshape(NG, S, L), s


# speedup vs baseline: 1.4786x; 1.0661x over previous
import jax
import jax.numpy as jnp
from jax import lax
from jax.experimental import pallas as pl
from jax.experimental.pallas import tpu as pltpu
from jax.experimental.pallas import tpu_sc as plsc

N = 10000
E = 320000
F = 128
H = 256
S = 32
NG = 8
L = 128

NC = 2
NS = 16
NW = NC * NS
HH = H // 2
EPW = E // NW
EPT = E // NS
C = 80
GCH = EPW // C
SCH = EPT // C
RPT = 624
ZR = 78

_SC_PARAMS = pltpu.CompilerParams(use_tc_tiling_on_sc=False,
                                  needs_layout_passes=False)


def _dg(x, w):
    return lax.dot_general(x, w, (((1,), (1,)), ((), ())),
                           preferred_element_type=jnp.float32)


def _make_gather(p):
    mesh = plsc.VectorSubcoreMesh(core_axis_name="c", subcore_axis_name="s")
    out_type = tuple(jax.ShapeDtypeStruct((E, HH), jnp.float32)
                     for _ in range(p))
    scratch = ([pltpu.VMEM((EPW,), jnp.int32) for _ in range(p)]
               + [pltpu.VMEM((C,), jnp.int32) for _ in range(p)]
               + [pltpu.VMEM((C, HH), jnp.float32) for _ in range(2 * p)]
               + [pltpu.SemaphoreType.DMA,
                  pltpu.SemaphoreType.DMA, pltpu.SemaphoreType.DMA])

    def body(table, *rest):
        idxs = rest[:p]
        outs = rest[p:2 * p]
        ivs = rest[2 * p:3 * p]
        cvs = rest[3 * p:4 * p]
        bufsets = (rest[4 * p:5 * p], rest[5 * p:6 * p])
        semg = rest[6 * p]
        semws = (rest[6 * p + 1], rest[6 * p + 2])
        cid = lax.axis_index("c")
        sid = lax.axis_index("s")
        base = (cid * NS + sid) * EPW
        for q in range(p):
            pltpu.sync_copy(idxs[q].at[pl.ds(base, EPW)], ivs[q])

        def do_chunk(cb, par):
            bufs = bufsets[par]
            for q in range(p):
                for k in range(C // 16):
                    cvs[q][pl.ds(k * 16, 16)] = ivs[q][pl.ds(cb + k * 16, 16)]
            gps = [pltpu.async_copy(table.at[cvs[q]], bufs[q], semg)
                   for q in range(p)]
            for gp in gps:
                gp.wait()
            for q in range(p):
                pltpu.async_copy(bufs[q], outs[q].at[pl.ds(base + cb, C)],
                                 semws[par])

        def drain_writes(par):
            for q in range(p):
                pltpu.make_async_copy(table.at[pl.ds(0, C)],
                                      bufsets[par][q], semws[par]).wait()

        def pair(j2, _):
            @pl.when(j2 > 0)
            def _():
                drain_writes(0)
            do_chunk(2 * j2 * C, 0)

            @pl.when(j2 > 0)
            def _():
                drain_writes(1)
            do_chunk((2 * j2 + 1) * C, 1)
            return 0

        npairs = GCH // 2
        lax.fori_loop(0, npairs, pair, 0)
        drain_writes(0)
        if GCH % 2:
            do_chunk((GCH - 1) * C, 0)
        drain_writes(1)
        if GCH % 2:
            drain_writes(0)

    return pl.kernel(body, out_type=out_type, mesh=mesh,
                     scratch_types=scratch, compiler_params=_SC_PARAMS)


_gather_x = _make_gather(2)
_gather_h = _make_gather(4)


def _make_scatter():
    mesh = plsc.VectorSubcoreMesh(core_axis_name="c", subcore_axis_name="s")
    out_type = (jax.ShapeDtypeStruct((NC * N, HH), jnp.float32),)
    scratch = [
        pltpu.VMEM((EPT,), jnp.int32),
        pltpu.VMEM((C,), jnp.int32),
        pltpu.VMEM((C,), jnp.int32),
        pltpu.VMEM((C, HH), jnp.float32),
        pltpu.VMEM((C, HH), jnp.float32),
        pltpu.VMEM((ZR, HH), jnp.float32),
        pltpu.VMEM_SHARED((N, HH), jnp.float32),
        pltpu.SemaphoreType.DMA,
        pltpu.SemaphoreType.DMA,
    ]

    def body(mlo, mhi, dst, acc_out, dst_v, dc0, dc1, buf0, buf1, zbuf,
             acc_sh, sem0, sem1):
        cid = lax.axis_index("c")
        sid = lax.axis_index("s")
        zvec = jnp.zeros((16,), jnp.float32)

        def zloop(i, _):
            for r in range(HH // 16):
                zbuf[i, pl.ds(r * 16, 16)] = zvec
            return 0

        lax.fori_loop(0, ZR, zloop, 0)
        row0 = sid * RPT
        for k in range(RPT // ZR):
            pltpu.sync_copy(zbuf, acc_sh.at[pl.ds(row0 + k * ZR, ZR)])

        tail0 = NS * RPT
        ntail = N - tail0

        @pl.when(sid == NS - 1)
        def _():
            pltpu.sync_copy(zbuf.at[pl.ds(0, ntail)],
                            acc_sh.at[pl.ds(tail0, ntail)])

        base = sid * EPT
        pltpu.sync_copy(dst.at[pl.ds(base, EPT)], dst_v)

        plsc.subcore_barrier()

        def fire_load(cb, buf, sem):
            @pl.when(cid == 0)
            def _():
                pltpu.async_copy(mlo.at[pl.ds(base + cb, C)], buf, sem)

            @pl.when(cid == 1)
            def _():
                pltpu.async_copy(mhi.at[pl.ds(base + cb, C)], buf, sem)

        def drain(buf, sem):
            pltpu.make_async_copy(mlo.at[pl.ds(0, C)], buf, sem).wait()

        def refresh(dc, cb):
            for k in range(C // 16):
                dc[pl.ds(k * 16, 16)] = dst_v[pl.ds(cb + k * 16, 16)]

        fire_load(0, buf0, sem0)

        def chunk2(j2, _):
            ca = 2 * j2 * C
            cb2 = (2 * j2 + 1) * C
            fire_load(cb2, buf1, sem1)
            drain(buf0, sem0)
            refresh(dc0, ca)
            pltpu.sync_copy(buf0, acc_sh.at[dc0], add=True)

            @pl.when(j2 < SCH // 2 - 1)
            def _():
                fire_load((2 * j2 + 2) * C, buf0, sem0)

            drain(buf1, sem1)
            refresh(dc1, cb2)
            pltpu.sync_copy(buf1, acc_sh.at[dc1], add=True)
            return 0

        lax.fori_loop(0, SCH // 2, chunk2, 0)

        plsc.subcore_barrier()

        pltpu.sync_copy(acc_sh.at[pl.ds(row0, RPT)],
                        acc_out.at[pl.ds(cid * N + row0, RPT)])

        @pl.when(sid == NS - 1)
        def _():
            pltpu.sync_copy(acc_sh.at[pl.ds(tail0, ntail)],
                            acc_out.at[pl.ds(cid * N + tail0, ntail)])

    return pl.kernel(body, out_type=out_type, mesh=mesh,
                     scratch_types=scratch, compiler_params=_SC_PARAMS)


_scatter = _make_scatter()


BE = 2000
GE = E // BE
BN = 1000
G = N // BN


def _m1_body(xi_ref, xj_ref, w1_ref, b1_ref, w2_ref, b2_ref, lo_ref, hi_ref):
    tmp = jnp.concatenate([xi_ref[...], xj_ref[...]], axis=1)
    h = jnp.maximum(_dg(tmp, w1_ref[...]) + b1_ref[...], 0.0)
    m = _dg(h, w2_ref[...]) + b2_ref[...]
    lo_ref[...] = m[:, :HH]
    hi_ref[...] = m[:, HH:]


def _m1(xi, xj, w1, b1, w2, b2):
    return pl.pallas_call(
        _m1_body,
        grid=(GE,),
        in_specs=[
            pl.BlockSpec((BE, F), lambda i: (i, 0)),
            pl.BlockSpec((BE, F), lambda i: (i, 0)),
            pl.BlockSpec((H, 2 * F), lambda i: (0, 0)),
            pl.BlockSpec((1, H), lambda i: (0, 0)),
            pl.BlockSpec((H, H), lambda i: (0, 0)),
            pl.BlockSpec((1, H), lambda i: (0, 0)),
        ],
        out_specs=[
            pl.BlockSpec((BE, HH), lambda i: (i, 0)),
            pl.BlockSpec((BE, HH), lambda i: (i, 0)),
        ],
        out_shape=[
            jax.ShapeDtypeStruct((E, HH), jnp.float32),
            jax.ShapeDtypeStruct((E, HH), jnp.float32),
        ],
    )(xi, xj, w1, b1, w2, b2)


def _m2_body(il_ref, ih_ref, jl_ref, jh_ref, w1_ref, b1_ref, w2_ref, b2_ref,
             lo_ref, hi_ref):
    tmp = jnp.maximum(jnp.concatenate(
        [il_ref[...], ih_ref[...], jl_ref[...], jh_ref[...]], axis=1), 0.0)
    h = jnp.maximum(_dg(tmp, w1_ref[...]) + b1_ref[...], 0.0)
    m = _dg(h, w2_ref[...]) + b2_ref[...]
    lo_ref[...] = m[:, :HH]
    hi_ref[...] = m[:, HH:]


def _m2(il, ih, jl, jh, w1, b1, w2, b2):
    return pl.pallas_call(
        _m2_body,
        grid=(GE,),
        in_specs=[
            pl.BlockSpec((BE, HH), lambda i: (i, 0)),
            pl.BlockSpec((BE, HH), lambda i: (i, 0)),
            pl.BlockSpec((BE, HH), lambda i: (i, 0)),
            pl.BlockSpec((BE, HH), lambda i: (i, 0)),
            pl.BlockSpec((H, 2 * H), lambda i: (0, 0)),
            pl.BlockSpec((1, H), lambda i: (0, 0)),
            pl.BlockSpec((H, H), lambda i: (0, 0)),
            pl.BlockSpec((1, H), lambda i: (0, 0)),
        ],
        out_specs=[
            pl.BlockSpec((BE, HH), lambda i: (i, 0)),
            pl.BlockSpec((BE, HH), lambda i: (i, 0)),
        ],
        out_shape=[
            jax.ShapeDtypeStruct((E, HH), jnp.float32),
            jax.ShapeDtypeStruct((E, HH), jnp.float32),
        ],
    )(il, ih, jl, jh, w1, b1, w2, b2)


def _tc3_body(q0_ref, q1_ref, batch_ref,
              pw1_ref, pb1_ref, pw2_ref, pb2_ref,
              ow1_ref, ob1_ref, ow2_ref, ob2_ref,
              s_ref, lat_ref, pacc):
    i = pl.program_id(0)
    h = jnp.maximum(jnp.concatenate([q0_ref[...], q1_ref[...]], axis=1), 0.0)
    t = jnp.maximum(_dg(h, pw1_ref[...]) + pb1_ref[...], 0.0)
    a = _dg(t, pw2_ref[...]) + pb2_ref[...]
    m = jnp.max(a, axis=-1, keepdims=True)
    ex = jnp.exp(a - m)
    sm = ex / jnp.sum(ex, axis=-1, keepdims=True)
    s_ref[...] = sm
    gi = lax.broadcasted_iota(jnp.int32, (BN, NG), 1)
    gm = (batch_ref[...] == gi).astype(jnp.float32)
    srep = jnp.concatenate([sm] * NG, axis=1)
    grep = jnp.concatenate(
        [jnp.broadcast_to(gm[:, g:g + 1], (BN, S)) for g in range(NG)],
        axis=1)
    w_assign = srep * grep

    @pl.when(i == 0)
    def _():
        pacc[...] = jnp.zeros((NG * S, H), jnp.float32)

    pacc[...] += lax.dot_general(w_assign, h, (((0,), (0,)), ((), ())),
                                 precision=lax.Precision.HIGHEST,
                                 preferred_element_type=jnp.float32)

    @pl.when(i == G - 1)
    def _():
        p = pacc[...]
        t2 = jnp.maximum(_dg(p, ow1_ref[...]) + ob1_ref[...], 0.0)
        lat_ref[...] = _dg(t2, ow2_ref[...]) + ob2_ref[...]


def _tc3(accf, batch2, pw1, pb1, pw2, pb2, ow1, ob1, ow2, ob2):
    return pl.pallas_call(
        _tc3_body,
        grid=(G,),
        in_specs=[
            pl.BlockSpec((BN, HH), lambda i: (i, 0)),
            pl.BlockSpec((BN, HH), lambda i: (G + i, 0)),
            pl.BlockSpec((BN, 1), lambda i: (i, 0)),
            pl.BlockSpec((H, H), lambda i: (0, 0)),
            pl.BlockSpec((1, H), lambda i: (0, 0)),
            pl.BlockSpec((S, H), lambda i: (0, 0)),
            pl.BlockSpec((1, S), lambda i: (0, 0)),
            pl.BlockSpec((H, H), lambda i: (0, 0)),
            pl.BlockSpec((1, H), lambda i: (0, 0)),
            pl.BlockSpec((L, H), lambda i: (0, 0)),
            pl.BlockSpec((1, L), lambda i: (0, 0)),
        ],
        out_specs=[
            pl.BlockSpec((BN, S), lambda i: (i, 0)),
            pl.BlockSpec((NG * S, L), lambda i: (0, 0)),
        ],
        out_shape=[
            jax.ShapeDtypeStruct((N, S), jnp.float32),
            jax.ShapeDtypeStruct((NG * S, L), jnp.float32),
        ],
        scratch_shapes=[pltpu.VMEM((NG * S, H), jnp.float32)],
    )(accf, accf, batch2, pw1, pb1, pw2, pb2, ow1, ob1, ow2, ob2)


def kernel(x, edge_index, batch,
           g1w1, g1b1, g1w2, g1b2,
           g2w1, g2b1, g2w2, g2b2,
           pw1, pb1, pw2, pb2,
           ow1, ob1, ow2, ob2):
    ei = edge_index.astype(jnp.int32)
    src = ei[0]
    dst = ei[1]
    srcN = src + N
    dstN = dst + N
    batch2 = batch.astype(jnp.int32).reshape(N, 1)

    xi, xj = _gather_x(x, dst, src)
    mlo1, mhi1 = _m1(xi, xj, g1w1, g1b1.reshape(1, H), g1w2, g1b2.reshape(1, H))
    (acc1,) = _scatter(mlo1, mhi1, dst)

    il, ih, jl, jh = _gather_h(acc1, dst, dstN, src, srcN)
    mlo2, mhi2 = _m2(il, ih, jl, jh, g2w1, g2b1.reshape(1, H),
                     g2w2, g2b2.reshape(1, H))
    (acc2,) = _scatter(mlo2, mhi2, dst)

    s, lat = _tc3(acc2, batch2,
                  pw1, pb1.reshape(1, H), pw2, pb2.reshape(1, S),
                  ow1, ob1.reshape(1, H), ow2, ob2.reshape(1, L))
    return lat.reshape(NG, S, L), s


# speedup vs baseline: 1.5126x; 1.0230x over previous
import jax
import jax.numpy as jnp
from jax import lax
from jax.experimental import pallas as pl
from jax.experimental.pallas import tpu as pltpu
from jax.experimental.pallas import tpu_sc as plsc

N = 10000
E = 320000
F = 128
H = 256
S = 32
NG = 8
L = 128

NC = 2
NS = 16
NW = NC * NS
HH = H // 2
EPW = E // NW
EPT = E // NS
C = 80
GCH = EPW // C
SCH = EPT // C
RPT = 624
ZR = 78

_SC_PARAMS = pltpu.CompilerParams(use_tc_tiling_on_sc=False,
                                  needs_layout_passes=False)


def _dg(x, w):
    return lax.dot_general(x, w, (((1,), (1,)), ((), ())),
                           preferred_element_type=jnp.float32)


def _make_gather(p):
    mesh = plsc.VectorSubcoreMesh(core_axis_name="c", subcore_axis_name="s")
    out_type = tuple(jax.ShapeDtypeStruct((E, HH), jnp.float32)
                     for _ in range(p))
    scratch = ([pltpu.VMEM((EPW,), jnp.int32) for _ in range(p)]
               + [pltpu.VMEM((C,), jnp.int32) for _ in range(2 * p)]
               + [pltpu.VMEM((C, HH), jnp.float32) for _ in range(2 * p)]
               + [pltpu.SemaphoreType.DMA, pltpu.SemaphoreType.DMA,
                  pltpu.SemaphoreType.DMA, pltpu.SemaphoreType.DMA])

    def body(table, *rest):
        idxs = rest[:p]
        outs = rest[p:2 * p]
        ivs = rest[2 * p:3 * p]
        cvsets = (rest[3 * p:4 * p], rest[4 * p:5 * p])
        bufsets = (rest[5 * p:6 * p], rest[6 * p:7 * p])
        semgs = (rest[7 * p], rest[7 * p + 1])
        semws = (rest[7 * p + 2], rest[7 * p + 3])
        cid = lax.axis_index("c")
        sid = lax.axis_index("s")
        base = (cid * NS + sid) * EPW
        for q in range(p):
            pltpu.sync_copy(idxs[q].at[pl.ds(base, EPW)], ivs[q])

        def fire_chunk(cb, par):
            cvs = cvsets[par]
            for q in range(p):
                for k in range(C // 16):
                    cvs[q][pl.ds(k * 16, 16)] = ivs[q][pl.ds(cb + k * 16, 16)]
            for q in range(p):
                pltpu.async_copy(table.at[cvs[q]], bufsets[par][q],
                                 semgs[par])

        def complete_chunk(cb, par):
            for q in range(p):
                pltpu.make_async_copy(table.at[pl.ds(0, C)],
                                      bufsets[par][q], semgs[par]).wait()
            for q in range(p):
                pltpu.async_copy(bufsets[par][q],
                                 outs[q].at[pl.ds(base + cb, C)], semws[par])

        def drain_writes(par):
            for q in range(p):
                pltpu.make_async_copy(table.at[pl.ds(0, C)],
                                      bufsets[par][q], semws[par]).wait()

        fire_chunk(0, 0)

        def pair(j2, _):
            fire_chunk((2 * j2 + 1) * C, 1)
            complete_chunk(2 * j2 * C, 0)

            @pl.when(j2 < GCH // 2 - 1)
            def _():
                drain_writes(0)
                fire_chunk((2 * j2 + 2) * C, 0)

            complete_chunk((2 * j2 + 1) * C, 1)

            @pl.when(j2 < GCH // 2 - 1)
            def _():
                drain_writes(1)
            return 0

        lax.fori_loop(0, GCH // 2, pair, 0)
        drain_writes(0)
        if GCH % 2:
            fire_chunk((GCH - 1) * C, 0)
            complete_chunk((GCH - 1) * C, 0)
            drain_writes(0)
        drain_writes(1)

    return pl.kernel(body, out_type=out_type, mesh=mesh,
                     scratch_types=scratch, compiler_params=_SC_PARAMS)


_gather_x = _make_gather(2)
_gather_h = _make_gather(4)


def _make_scatter():
    mesh = plsc.VectorSubcoreMesh(core_axis_name="c", subcore_axis_name="s")
    out_type = (jax.ShapeDtypeStruct((NC * N, HH), jnp.float32),)
    scratch = [
        pltpu.VMEM((EPT,), jnp.int32),
        pltpu.VMEM((C,), jnp.int32),
        pltpu.VMEM((C,), jnp.int32),
        pltpu.VMEM((C, HH), jnp.float32),
        pltpu.VMEM((C, HH), jnp.float32),
        pltpu.VMEM((ZR, HH), jnp.float32),
        pltpu.VMEM_SHARED((N, HH), jnp.float32),
        pltpu.SemaphoreType.DMA,
        pltpu.SemaphoreType.DMA,
    ]

    def body(mlo, mhi, dst, acc_out, dst_v, dc0, dc1, buf0, buf1, zbuf,
             acc_sh, sem0, sem1):
        cid = lax.axis_index("c")
        sid = lax.axis_index("s")
        zvec = jnp.zeros((16,), jnp.float32)

        def zloop(i, _):
            for r in range(HH // 16):
                zbuf[i, pl.ds(r * 16, 16)] = zvec
            return 0

        lax.fori_loop(0, ZR, zloop, 0)
        row0 = sid * RPT
        for k in range(RPT // ZR):
            pltpu.sync_copy(zbuf, acc_sh.at[pl.ds(row0 + k * ZR, ZR)])

        tail0 = NS * RPT
        ntail = N - tail0

        @pl.when(sid == NS - 1)
        def _():
            pltpu.sync_copy(zbuf.at[pl.ds(0, ntail)],
                            acc_sh.at[pl.ds(tail0, ntail)])

        base = sid * EPT
        pltpu.sync_copy(dst.at[pl.ds(base, EPT)], dst_v)

        plsc.subcore_barrier()

        def fire_load(cb, buf, sem):
            @pl.when(cid == 0)
            def _():
                pltpu.async_copy(mlo.at[pl.ds(base + cb, C)], buf, sem)

            @pl.when(cid == 1)
            def _():
                pltpu.async_copy(mhi.at[pl.ds(base + cb, C)], buf, sem)

        def drain(buf, sem):
            pltpu.make_async_copy(mlo.at[pl.ds(0, C)], buf, sem).wait()

        def refresh(dc, cb):
            for k in range(C // 16):
                dc[pl.ds(k * 16, 16)] = dst_v[pl.ds(cb + k * 16, 16)]

        fire_load(0, buf0, sem0)

        def chunk2(j2, _):
            ca = 2 * j2 * C
            cb2 = (2 * j2 + 1) * C
            fire_load(cb2, buf1, sem1)
            drain(buf0, sem0)
            refresh(dc0, ca)
            pltpu.sync_copy(buf0, acc_sh.at[dc0], add=True)

            @pl.when(j2 < SCH // 2 - 1)
            def _():
                fire_load((2 * j2 + 2) * C, buf0, sem0)

            drain(buf1, sem1)
            refresh(dc1, cb2)
            pltpu.sync_copy(buf1, acc_sh.at[dc1], add=True)
            return 0

        lax.fori_loop(0, SCH // 2, chunk2, 0)

        plsc.subcore_barrier()

        pltpu.sync_copy(acc_sh.at[pl.ds(row0, RPT)],
                        acc_out.at[pl.ds(cid * N + row0, RPT)])

        @pl.when(sid == NS - 1)
        def _():
            pltpu.sync_copy(acc_sh.at[pl.ds(tail0, ntail)],
                            acc_out.at[pl.ds(cid * N + tail0, ntail)])

    return pl.kernel(body, out_type=out_type, mesh=mesh,
                     scratch_types=scratch, compiler_params=_SC_PARAMS)


_scatter = _make_scatter()


BE = 2000
GE = E // BE
BN = 1000
G = N // BN


def _m1_body(xi_ref, xj_ref, w1_ref, b1_ref, w2_ref, b2_ref, lo_ref, hi_ref):
    tmp = jnp.concatenate([xi_ref[...], xj_ref[...]], axis=1)
    h = jnp.maximum(_dg(tmp, w1_ref[...]) + b1_ref[...], 0.0)
    m = _dg(h, w2_ref[...]) + b2_ref[...]
    lo_ref[...] = m[:, :HH]
    hi_ref[...] = m[:, HH:]


def _m1(xi, xj, w1, b1, w2, b2):
    return pl.pallas_call(
        _m1_body,
        grid=(GE,),
        in_specs=[
            pl.BlockSpec((BE, F), lambda i: (i, 0)),
            pl.BlockSpec((BE, F), lambda i: (i, 0)),
            pl.BlockSpec((H, 2 * F), lambda i: (0, 0)),
            pl.BlockSpec((1, H), lambda i: (0, 0)),
            pl.BlockSpec((H, H), lambda i: (0, 0)),
            pl.BlockSpec((1, H), lambda i: (0, 0)),
        ],
        out_specs=[
            pl.BlockSpec((BE, HH), lambda i: (i, 0)),
            pl.BlockSpec((BE, HH), lambda i: (i, 0)),
        ],
        out_shape=[
            jax.ShapeDtypeStruct((E, HH), jnp.float32),
            jax.ShapeDtypeStruct((E, HH), jnp.float32),
        ],
    )(xi, xj, w1, b1, w2, b2)


def _m2_body(il_ref, ih_ref, jl_ref, jh_ref, w1_ref, b1_ref, w2_ref, b2_ref,
             lo_ref, hi_ref):
    tmp = jnp.maximum(jnp.concatenate(
        [il_ref[...], ih_ref[...], jl_ref[...], jh_ref[...]], axis=1), 0.0)
    h = jnp.maximum(_dg(tmp, w1_ref[...]) + b1_ref[...], 0.0)
    m = _dg(h, w2_ref[...]) + b2_ref[...]
    lo_ref[...] = m[:, :HH]
    hi_ref[...] = m[:, HH:]


def _m2(il, ih, jl, jh, w1, b1, w2, b2):
    return pl.pallas_call(
        _m2_body,
        grid=(GE,),
        in_specs=[
            pl.BlockSpec((BE, HH), lambda i: (i, 0)),
            pl.BlockSpec((BE, HH), lambda i: (i, 0)),
            pl.BlockSpec((BE, HH), lambda i: (i, 0)),
            pl.BlockSpec((BE, HH), lambda i: (i, 0)),
            pl.BlockSpec((H, 2 * H), lambda i: (0, 0)),
            pl.BlockSpec((1, H), lambda i: (0, 0)),
            pl.BlockSpec((H, H), lambda i: (0, 0)),
            pl.BlockSpec((1, H), lambda i: (0, 0)),
        ],
        out_specs=[
            pl.BlockSpec((BE, HH), lambda i: (i, 0)),
            pl.BlockSpec((BE, HH), lambda i: (i, 0)),
        ],
        out_shape=[
            jax.ShapeDtypeStruct((E, HH), jnp.float32),
            jax.ShapeDtypeStruct((E, HH), jnp.float32),
        ],
    )(il, ih, jl, jh, w1, b1, w2, b2)


def _tc3_body(q0_ref, q1_ref, batch_ref,
              pw1_ref, pb1_ref, pw2_ref, pb2_ref,
              ow1_ref, ob1_ref, ow2_ref, ob2_ref,
              s_ref, lat_ref, pacc):
    i = pl.program_id(0)
    h = jnp.maximum(jnp.concatenate([q0_ref[...], q1_ref[...]], axis=1), 0.0)
    t = jnp.maximum(_dg(h, pw1_ref[...]) + pb1_ref[...], 0.0)
    a = _dg(t, pw2_ref[...]) + pb2_ref[...]
    m = jnp.max(a, axis=-1, keepdims=True)
    ex = jnp.exp(a - m)
    sm = ex / jnp.sum(ex, axis=-1, keepdims=True)
    s_ref[...] = sm
    gi = lax.broadcasted_iota(jnp.int32, (BN, NG), 1)
    gm = (batch_ref[...] == gi).astype(jnp.float32)
    srep = jnp.concatenate([sm] * NG, axis=1)
    grep = jnp.concatenate(
        [jnp.broadcast_to(gm[:, g:g + 1], (BN, S)) for g in range(NG)],
        axis=1)
    w_assign = srep * grep

    @pl.when(i == 0)
    def _():
        pacc[...] = jnp.zeros((NG * S, H), jnp.float32)

    pacc[...] += lax.dot_general(w_assign, h, (((0,), (0,)), ((), ())),
                                 precision=lax.Precision.HIGHEST,
                                 preferred_element_type=jnp.float32)

    @pl.when(i == G - 1)
    def _():
        p = pacc[...]
        t2 = jnp.maximum(_dg(p, ow1_ref[...]) + ob1_ref[...], 0.0)
        lat_ref[...] = _dg(t2, ow2_ref[...]) + ob2_ref[...]


def _tc3(accf, batch2, pw1, pb1, pw2, pb2, ow1, ob1, ow2, ob2):
    return pl.pallas_call(
        _tc3_body,
        grid=(G,),
        in_specs=[
            pl.BlockSpec((BN, HH), lambda i: (i, 0)),
            pl.BlockSpec((BN, HH), lambda i: (G + i, 0)),
            pl.BlockSpec((BN, 1), lambda i: (i, 0)),
            pl.BlockSpec((H, H), lambda i: (0, 0)),
            pl.BlockSpec((1, H), lambda i: (0, 0)),
            pl.BlockSpec((S, H), lambda i: (0, 0)),
            pl.BlockSpec((1, S), lambda i: (0, 0)),
            pl.BlockSpec((H, H), lambda i: (0, 0)),
            pl.BlockSpec((1, H), lambda i: (0, 0)),
            pl.BlockSpec((L, H), lambda i: (0, 0)),
            pl.BlockSpec((1, L), lambda i: (0, 0)),
        ],
        out_specs=[
            pl.BlockSpec((BN, S), lambda i: (i, 0)),
            pl.BlockSpec((NG * S, L), lambda i: (0, 0)),
        ],
        out_shape=[
            jax.ShapeDtypeStruct((N, S), jnp.float32),
            jax.ShapeDtypeStruct((NG * S, L), jnp.float32),
        ],
        scratch_shapes=[pltpu.VMEM((NG * S, H), jnp.float32)],
    )(accf, accf, batch2, pw1, pb1, pw2, pb2, ow1, ob1, ow2, ob2)


def kernel(x, edge_index, batch,
           g1w1, g1b1, g1w2, g1b2,
           g2w1, g2b1, g2w2, g2b2,
           pw1, pb1, pw2, pb2,
           ow1, ob1, ow2, ob2):
    ei = edge_index.astype(jnp.int32)
    src = ei[0]
    dst = ei[1]
    srcN = src + N
    dstN = dst + N
    batch2 = batch.astype(jnp.int32).reshape(N, 1)

    xi, xj = _gather_x(x, dst, src)
    mlo1, mhi1 = _m1(xi, xj, g1w1, g1b1.reshape(1, H), g1w2, g1b2.reshape(1, H))
    (acc1,) = _scatter(mlo1, mhi1, dst)

    il, ih, jl, jh = _gather_h(acc1, dst, dstN, src, srcN)
    mlo2, mhi2 = _m2(il, ih, jl, jh, g2w1, g2b1.reshape(1, H),
                     g2w2, g2b2.reshape(1, H))
    (acc2,) = _scatter(mlo2, mhi2, dst)

    s, lat = _tc3(acc2, batch2,
                  pw1, pb1.reshape(1, H), pw2, pb2.reshape(1, S),
                  ow1, ob1.reshape(1, H), ow2, ob2.reshape(1, L))
    return lat.reshape(NG, S, L), s
